# Initial kernel scaffold; baseline (speedup 1.0000x reference)
import jax
import jax.numpy as jnp
from jax.experimental import pallas as pl


def kernel(c1w, c1b, c2w, c2b, c3w, c3b, w1, b1, w2, b2, x):
    raise NotImplementedError("write your pallas kernel here")


# speedup vs baseline: 81.4734x; 81.4734x over previous
import functools

import jax
import jax.numpy as jnp
from jax.experimental import pallas as pl
from jax.experimental.pallas import tpu as pltpu

_NB = 16


def _fused_qnet_kernel(xs_ref, c1w_ref, c1b_ref, c2w_ref, c2b_ref,
                       c3w_ref, c3b_ref, w1_ref, b1_ref, w2_ref, b2_ref,
                       o_ref, flat_ref):
    nb = xs_ref.shape[0]
    xs = xs_ref[...]

    p1 = jnp.concatenate(
        [xs[:, i:i + 20, j:j + 20, :] for i in range(2) for j in range(2)],
        axis=-1).reshape(nb * 400, 256)
    a1 = jnp.dot(p1, c1w_ref[...], preferred_element_type=jnp.float32)
    a1 = jnp.maximum(a1 + c1b_ref[...], 0.0)
    h1 = a1.astype(jnp.bfloat16)[:, :32].reshape(nb, 10, 2, 10, 2, 32)

    cols2 = []
    for i in range(4):
        ai, pi = divmod(i, 2)
        for j in range(4):
            aj, pj = divmod(j, 2)
            cols2.append(h1[:, ai:ai + 9, pi, aj:aj + 9, pj, :])
    p2 = jnp.concatenate(cols2, axis=-1).reshape(nb * 81, 512)
    a2 = jnp.dot(p2, c2w_ref[...], preferred_element_type=jnp.float32)
    a2 = jnp.maximum(a2 + c2b_ref[...], 0.0)
    h2 = a2.astype(jnp.bfloat16)[:, :64].reshape(nb, 9, 9, 64)

    p3 = jnp.concatenate(
        [h2[:, i:i + 7, j:j + 7, :] for i in range(3) for j in range(3)],
        axis=-1).reshape(nb * 49, 576)
    a3 = jnp.dot(p3, c3w_ref[...], preferred_element_type=jnp.float32)
    a3 = jnp.maximum(a3 + c3b_ref[...], 0.0)
    h3v = a3.astype(jnp.bfloat16)[:, :64].reshape(nb, 49, 64)
    for p in range(49):
        flat_ref[:, p * 64:(p + 1) * 64] = h3v[:, p, :]
    h3 = flat_ref[...]

    hid = jnp.dot(h3, w1_ref[...], preferred_element_type=jnp.float32)
    hid = jnp.maximum(hid + b1_ref[...], 0.0).astype(jnp.bfloat16)
    q = jnp.dot(hid, w2_ref[...], preferred_element_type=jnp.float32)
    o_ref[...] = q + b2_ref[...]


def _resident(shape):
    nd = len(shape)
    return pl.BlockSpec(shape, lambda i: (0,) * nd)


@jax.jit
def kernel(c1w, c1b, c2w, c2b, c3w, c3b, w1, b1, w2, b2, x):
    B = x.shape[0]
    nb = _NB

    xs = jnp.transpose(x, (0, 2, 3, 1)).astype(jnp.bfloat16)
    xs = xs.reshape(B, 21, 4, 21, 4, 4).transpose(0, 1, 3, 2, 4, 5)
    xs = xs.reshape(B, 21, 21, 64)

    c1w_s = c1w.reshape(2, 4, 2, 4, 4, c1w.shape[1])
    c1w_s = c1w_s.transpose(0, 2, 1, 3, 4, 5).reshape(256, c1w.shape[1])

    out = pl.pallas_call(
        _fused_qnet_kernel,
        out_shape=jax.ShapeDtypeStruct((B, 128), jnp.float32),
        grid=(B // nb,),
        in_specs=[
            pl.BlockSpec((nb, 21, 21, 64), lambda i: (i, 0, 0, 0)),
            _resident(c1w_s.shape),
            _resident(c1b.shape),
            _resident(c2w.shape),
            _resident(c2b.shape),
            _resident(c3w.shape),
            _resident(c3b.shape),
            _resident(w1.shape),
            _resident(b1.shape),
            _resident(w2.shape),
            _resident(b2.shape),
        ],
        out_specs=pl.BlockSpec((nb, 128), lambda i: (i, 0)),
        scratch_shapes=[pltpu.VMEM((nb, 3136), jnp.bfloat16)],
        compiler_params=pltpu.CompilerParams(
            dimension_semantics=("parallel",)),
    )(xs, c1w_s, c1b, c2w, c2b, c3w, c3b, w1, b1, w2, b2)
    return out[:, :4]
